# Initial kernel scaffold; baseline (speedup 1.0000x reference)
import jax
import jax.numpy as jnp
from jax.experimental import pallas as pl


def kernel(z_e, embedding):
    raise NotImplementedError("write your pallas kernel here")


# speedup vs baseline: 1.6452x; 1.6452x over previous
import functools

import jax
import jax.numpy as jnp
from jax.experimental import pallas as pl
from jax.experimental.pallas import tpu as pltpu

_BR = 1024


def _vq_body(z_ref, emb_ref, idx_ref, q_ref, loss_ref, *, n_codes):
    pid = pl.program_id(0)
    zb = z_ref[...]
    emb = emb_ref[...]
    z2 = jnp.sum(zb * zb, axis=1, keepdims=True)
    e2 = jnp.sum(emb * emb, axis=1)
    mm = jax.lax.dot_general(zb, emb, (((1,), (1,)), ((), ())),
                             preferred_element_type=jnp.float32)
    d = z2 + e2[None, :] - 2.0 * mm
    idx = jnp.argmin(d, axis=1).astype(jnp.int32)
    dmin = jnp.min(d, axis=1)
    onehot = (jax.lax.broadcasted_iota(jnp.int32, d.shape, 1)
              == idx[:, None]).astype(jnp.float32)
    qb = jax.lax.dot_general(onehot, emb, (((1,), (0,)), ((), ())),
                             preferred_element_type=jnp.float32)
    idx_ref[0, 0, :] = idx
    q_ref[...] = qb

    @pl.when(pid == 0)
    def _init():
        loss_ref[...] = jnp.zeros_like(loss_ref)

    loss_ref[...] += jnp.sum(dmin).reshape(1, 1)


def kernel(z_e, embedding):
    batch, ch, w, h = z_e.shape
    n_codes = embedding.shape[0]
    n_rows = batch * w * h
    nb = n_rows // _BR
    z_flat = jnp.transpose(z_e, (0, 2, 3, 1)).reshape(n_rows, ch)

    idx3, q_flat, loss_sum = pl.pallas_call(
        functools.partial(_vq_body, n_codes=n_codes),
        grid=(nb,),
        in_specs=[
            pl.BlockSpec((_BR, ch), lambda i: (i, 0)),
            pl.BlockSpec((n_codes, ch), lambda i: (0, 0)),
        ],
        out_specs=[
            pl.BlockSpec((1, 1, _BR), lambda i: (i, 0, 0)),
            pl.BlockSpec((_BR, ch), lambda i: (i, 0)),
            pl.BlockSpec((1, 1), lambda i: (0, 0)),
        ],
        out_shape=[
            jax.ShapeDtypeStruct((nb, 1, _BR), jnp.int32),
            jax.ShapeDtypeStruct((n_rows, ch), jnp.float32),
            jax.ShapeDtypeStruct((1, 1), jnp.float32),
        ],
    )(z_flat, embedding)

    indices = idx3.reshape(n_rows)
    quantized_out = jnp.transpose(q_flat.reshape(batch, w, h, ch), (0, 3, 1, 2))
    vq_loss = loss_sum[0, 0] * (1.25 / (n_rows * ch))
    return quantized_out, indices, vq_loss


# speedup vs baseline: 2.2665x; 1.3776x over previous
import functools

import jax
import jax.numpy as jnp
from jax.experimental import pallas as pl
from jax.experimental.pallas import tpu as pltpu


def _vq_body(z_ref, emb_ref, idx_ref, q_ref, loss_ref):
    zbt = z_ref[0]
    emb = emb_ref[...]
    e2 = jnp.sum(emb * emb, axis=1)
    mmt = jax.lax.dot_general(emb, zbt, (((1,), (0,)), ((), ())),
                              preferred_element_type=jnp.float32)
    dt = e2[:, None] - 2.0 * mmt
    idx = jnp.argmin(dt, axis=0).astype(jnp.int32)
    dmin = jnp.min(dt, axis=0)
    z2 = jnp.sum(zbt * zbt, axis=0)
    onehot = (jax.lax.broadcasted_iota(jnp.int32, dt.shape, 0)
              == idx[None, :]).astype(jnp.float32)
    qt = jax.lax.dot_general(emb, onehot, (((0,), (0,)), ((), ())),
                             preferred_element_type=jnp.float32)
    idx_ref[0, 0, :] = idx
    q_ref[0] = qt
    loss_ref[...] = jnp.sum(dmin + z2).reshape(1, 1, 1)


def kernel(z_e, embedding):
    batch, ch, w, h = z_e.shape
    n_codes = embedding.shape[0]
    pix = w * h
    z3 = z_e.reshape(batch, ch, pix)

    idx3, q3, loss_parts = pl.pallas_call(
        _vq_body,
        grid=(batch,),
        in_specs=[
            pl.BlockSpec((1, ch, pix), lambda i: (i, 0, 0)),
            pl.BlockSpec((n_codes, ch), lambda i: (0, 0)),
        ],
        out_specs=[
            pl.BlockSpec((1, 1, pix), lambda i: (i, 0, 0)),
            pl.BlockSpec((1, ch, pix), lambda i: (i, 0, 0)),
            pl.BlockSpec((1, 1, 1), lambda i: (i, 0, 0)),
        ],
        out_shape=[
            jax.ShapeDtypeStruct((batch, 1, pix), jnp.int32),
            jax.ShapeDtypeStruct((batch, ch, pix), jnp.float32),
            jax.ShapeDtypeStruct((batch, 1, 1), jnp.float32),
        ],
        compiler_params=pltpu.CompilerParams(
            dimension_semantics=("parallel",)),
    )(z3, embedding)

    indices = idx3.reshape(batch * pix)
    quantized_out = q3.reshape(batch, ch, w, h)
    vq_loss = jnp.sum(loss_parts) * (1.25 / (batch * pix * ch))
    return quantized_out, indices, vq_loss
